# Initial kernel scaffold; baseline (speedup 1.0000x reference)
import jax
import jax.numpy as jnp
from jax.experimental import pallas as pl


def kernel(pred, target, weight):
    raise NotImplementedError("write your pallas kernel here")


# speedup vs baseline: 1.9835x; 1.9835x over previous
import functools

import jax
import jax.numpy as jnp
from jax import lax
from jax.experimental import pallas as pl
from jax.experimental.pallas import tpu as pltpu
from jax.experimental.pallas import tpu_sc as plsc

_MU = 0.02
_BINS = 10
_LOSS_WEIGHT = 1.0

_N = 2_000_000
_C = 8_000
_NCHUNKS = _N // _C
_VPC = _C // 16
_NW = 32


def _sc_histogram_pass(pred_flat, target_flat, weight_flat):
    mesh = plsc.VectorSubcoreMesh(core_axis_name="c", subcore_axis_name="s")

    @functools.partial(
        pl.kernel,
        mesh=mesh,
        out_type=(
            jax.ShapeDtypeStruct((_NW, _BINS * 16), jnp.float32),
            jax.ShapeDtypeStruct((_NW, _BINS * 16), jnp.float32),
            jax.ShapeDtypeStruct((_NW, 16), jnp.float32),
        ),
        scratch_types=[
            pltpu.VMEM((_C,), jnp.float32),
            pltpu.VMEM((_C,), jnp.float32),
            pltpu.VMEM((_C,), jnp.float32),
            pltpu.VMEM((_BINS * 16,), jnp.float32),
            pltpu.VMEM((_BINS * 16,), jnp.float32),
            pltpu.VMEM((16,), jnp.float32),
        ],
        compiler_params=pltpu.CompilerParams(needs_layout_passes=False),
    )
    def k(pred_hbm, target_hbm, weight_hbm, cnt_hbm, sum_hbm, tw_hbm,
          pbuf, tbuf, wbuf, cnt_h, sum_h, tw_buf):
        wid = lax.axis_index("s") * 2 + lax.axis_index("c")
        zero16 = jnp.zeros((16,), jnp.float32)
        for b in range(_BINS):
            cnt_h[pl.ds(b * 16, 16)] = zero16
            sum_h[pl.ds(b * 16, 16)] = zero16

        lane = lax.iota(jnp.int32, 16)
        mu = jnp.float32(_MU)
        mu2 = jnp.float32(_MU * _MU)
        nchunks = (jnp.int32(_NCHUNKS) - wid + (_NW - 1)) // _NW

        def chunk_body(ci, tacc):
            off = (wid + ci * _NW) * _C
            pltpu.sync_copy(pred_hbm.at[pl.ds(off, _C)], pbuf)
            pltpu.sync_copy(target_hbm.at[pl.ds(off, _C)], tbuf)
            pltpu.sync_copy(weight_hbm.at[pl.ds(off, _C)], wbuf)

            def vreg_body(i, acc):
                sl = pl.ds(i * 16, 16)
                p = pbuf[sl]
                t = tbuf[sl]
                w = wbuf[sl]
                d = p - t
                s = d * d + mu2
                ibits = lax.bitcast_convert_type(s, jnp.int32)
                seed = jnp.int32(0x5F3759DF) - lax.shift_right_logical(ibits, 1)
                y = lax.bitcast_convert_type(seed, jnp.float32)
                sh = jnp.float32(0.5) * s
                y = y * (jnp.float32(1.5) - sh * y * y)
                y = y * (jnp.float32(1.5) - sh * y * y)
                loss = s * y - mu
                g = jnp.abs(d) * y
                validf = jnp.where(w > 0, jnp.float32(1.0), jnp.float32(0.0))
                lwv = loss * w * validf
                b = jnp.minimum((g * jnp.float32(10.0)).astype(jnp.int32), 9)
                idx = b * 16 + lane
                plsc.addupdate_scatter(cnt_h, [idx], validf)
                plsc.addupdate_scatter(sum_h, [idx], lwv)
                return acc + w

            return lax.fori_loop(0, _VPC, vreg_body, tacc)

        tacc = lax.fori_loop(0, nchunks, chunk_body, zero16)

        tw_buf[...] = tacc
        pltpu.sync_copy(cnt_h, cnt_hbm.at[wid])
        pltpu.sync_copy(sum_h, sum_hbm.at[wid])
        pltpu.sync_copy(tw_buf, tw_hbm.at[wid])

    return k(pred_flat, target_flat, weight_flat)


def _epilogue_body(cnt_ref, sum_ref, tw_ref, o_ref):
    tot = jnp.maximum(jnp.sum(tw_ref[...]), 1.0)
    r = jnp.float32(0.0)
    nbins = jnp.float32(0.0)
    for b in range(_BINS):
        cb = jnp.sum(cnt_ref[:, b * 16:(b + 1) * 16])
        sb = jnp.sum(sum_ref[:, b * 16:(b + 1) * 16])
        pos = cb > 0
        nbins = nbins + jnp.where(pos, 1.0, 0.0)
        r = r + jnp.where(pos, (tot / jnp.maximum(cb, 1.0)) * sb, 0.0)
    r = r / jnp.maximum(nbins, 1.0)
    o_ref[0, 0] = r * jnp.float32(_LOSS_WEIGHT / _N)


def kernel(pred, target, weight):
    cnt, s, tw = _sc_histogram_pass(
        pred.reshape(-1), target.reshape(-1), weight.reshape(-1))
    out = pl.pallas_call(
        _epilogue_body,
        out_shape=jax.ShapeDtypeStruct((1, 1), jnp.float32),
        out_specs=pl.BlockSpec(memory_space=pltpu.SMEM),
    )(cnt, s, tw)
    return out[0, 0]


# speedup vs baseline: 4.2847x; 2.1602x over previous
import functools

import jax
import jax.numpy as jnp
from jax import lax
from jax.experimental import pallas as pl
from jax.experimental.pallas import tpu as pltpu
from jax.experimental.pallas import tpu_sc as plsc

_MU = 0.02
_BINS = 10
_LOSS_WEIGHT = 1.0

_N = 2_000_000
_C = 800
_R = _C // 4
_NCHUNKS = _N // _C
_VPC = _C // 16
_NW = 32


def _sc_histogram_pass(pred_flat, target_flat, weight_flat):
    mesh = plsc.VectorSubcoreMesh(core_axis_name="c", subcore_axis_name="s")

    @functools.partial(
        pl.kernel,
        mesh=mesh,
        out_type=(
            jax.ShapeDtypeStruct((_NW, _BINS * 16), jnp.float32),
            jax.ShapeDtypeStruct((_NW, _BINS * 16), jnp.float32),
            jax.ShapeDtypeStruct((_NW, 16), jnp.float32),
        ),
        scratch_types=[
            pltpu.VMEM((_R, 4), jnp.float32),
            pltpu.VMEM((_R, 4), jnp.float32),
            pltpu.VMEM((_R, 4), jnp.float32),
            pltpu.VMEM((_BINS * 16,), jnp.float32),
            pltpu.VMEM((_BINS * 16,), jnp.float32),
            pltpu.VMEM((16,), jnp.float32),
        ],
        compiler_params=pltpu.CompilerParams(needs_layout_passes=False),
    )
    def k(pred_hbm, target_hbm, weight_hbm, cnt_hbm, sum_hbm, tw_hbm,
          pbuf, tbuf, wbuf, cnt_h, sum_h, tw_buf):
        wid = lax.axis_index("s") * 2 + lax.axis_index("c")
        zero16 = jnp.zeros((16,), jnp.float32)
        for b in range(_BINS):
            cnt_h[pl.ds(b * 16, 16)] = zero16
            sum_h[pl.ds(b * 16, 16)] = zero16

        lane = lax.iota(jnp.int32, 16)
        rowpat = lax.shift_right_logical(lane, 2)
        colpat = lax.bitwise_and(lane, 3)
        mu = jnp.float32(_MU)
        mu2 = jnp.float32(_MU * _MU)
        nchunks = (jnp.int32(_NCHUNKS) - wid + (_NW - 1)) // _NW

        def chunk_body(ci, tacc):
            roff = (wid + ci * _NW) * _R
            pltpu.sync_copy(pred_hbm.at[pl.ds(roff, _R), :], pbuf)
            pltpu.sync_copy(target_hbm.at[pl.ds(roff, _R), :], tbuf)
            pltpu.sync_copy(weight_hbm.at[pl.ds(roff, _R), :], wbuf)

            def vreg_body(i, acc):
                ridx = i * 4 + rowpat
                p = plsc.load_gather(pbuf, [ridx, colpat])
                t = plsc.load_gather(tbuf, [ridx, colpat])
                w = plsc.load_gather(wbuf, [ridx, colpat])
                d = p - t
                s = d * d + mu2
                ibits = lax.bitcast_convert_type(s, jnp.int32)
                seed = jnp.int32(0x5F3759DF) - lax.shift_right_logical(ibits, 1)
                y = lax.bitcast_convert_type(seed, jnp.float32)
                sh = jnp.float32(0.5) * s
                y = y * (jnp.float32(1.5) - sh * y * y)
                y = y * (jnp.float32(1.5) - sh * y * y)
                loss = s * y - mu
                g = jnp.abs(d) * y
                validf = jnp.where(w > 0, jnp.float32(1.0), jnp.float32(0.0))
                lwv = loss * w * validf
                b = jnp.minimum((g * jnp.float32(10.0)).astype(jnp.int32), 9)
                idx = b * 16 + lane
                plsc.addupdate_scatter(cnt_h, [idx], validf)
                plsc.addupdate_scatter(sum_h, [idx], lwv)
                return acc + w

            return lax.fori_loop(0, _VPC, vreg_body, tacc)

        tacc = lax.fori_loop(0, nchunks, chunk_body, zero16)

        tw_buf[...] = tacc
        pltpu.sync_copy(cnt_h, cnt_hbm.at[wid])
        pltpu.sync_copy(sum_h, sum_hbm.at[wid])
        pltpu.sync_copy(tw_buf, tw_hbm.at[wid])

    return k(pred_flat, target_flat, weight_flat)


def _epilogue_body(cnt_ref, sum_ref, tw_ref, o_ref):
    tot = jnp.maximum(jnp.sum(tw_ref[...]), 1.0)
    r = jnp.float32(0.0)
    nbins = jnp.float32(0.0)
    for b in range(_BINS):
        cb = jnp.sum(cnt_ref[:, b * 16:(b + 1) * 16])
        sb = jnp.sum(sum_ref[:, b * 16:(b + 1) * 16])
        pos = cb > 0
        nbins = nbins + jnp.where(pos, 1.0, 0.0)
        r = r + jnp.where(pos, (tot / jnp.maximum(cb, 1.0)) * sb, 0.0)
    r = r / jnp.maximum(nbins, 1.0)
    o_ref[0, 0] = r * jnp.float32(_LOSS_WEIGHT / _N)


def kernel(pred, target, weight):
    cnt, s, tw = _sc_histogram_pass(pred, target, weight)
    out = pl.pallas_call(
        _epilogue_body,
        out_shape=jax.ShapeDtypeStruct((1, 1), jnp.float32),
        out_specs=pl.BlockSpec(memory_space=pltpu.SMEM),
    )(cnt, s, tw)
    return out[0, 0]
